# baseline (speedup 1.0000x reference)
import jax
import jax.numpy as jnp
from jax.experimental import pallas as pl
from jax.experimental.pallas import tpu as pltpu

N = 4096
DIM = 128
L = 3
CH = 256
BLK = 1024


def _unet_kernel(g_hbm, h_ref, wd_ref, bd_ref, wu_ref, bu_ref, wb_ref, bb_ref,
                 o0_ref, o1_ref, o2_ref, o3_ref,
                 gb_ref, stage_ref, zb_ref, t0_ref, t1_ref, t2_ref,
                 p0_ref, p1_ref, sem):

    def g_dma(i, slot):
        return pltpu.make_async_copy(
            g_hbm.at[pl.ds(i * CH, CH), :], stage_ref.at[slot], sem.at[slot])

    zb_ref[...] = jnp.dot(
        h_ref[...].astype(jnp.bfloat16), wd_ref[0].astype(jnp.bfloat16),
        preferred_element_type=jnp.float32).astype(jnp.bfloat16)
    b0 = bd_ref[0]

    g_dma(0, 0).start()
    for i in range(N // CH):
        slot = i % 2
        if i + 1 < N // CH:
            g_dma(i + 1, 1 - slot).start()
        g_dma(i, slot).wait()
        gchunk = stage_ref[slot].astype(jnp.bfloat16)
        gb_ref[pl.ds(i * CH, CH), :] = gchunk
        y = jax.nn.relu(
            jnp.dot(gchunk, zb_ref[...], preferred_element_type=jnp.float32)
            + b0[None, :])
        t0_ref[pl.ds(i * CH, CH), :] = y.astype(jnp.bfloat16)

    def layer(x_ref, W, b, store_ref=None, skip_ref=None, f32_ref=None,
              final=False):
        zb_ref[...] = jnp.dot(
            x_ref[...], W.astype(jnp.bfloat16),
            preferred_element_type=jnp.float32).astype(jnp.bfloat16)

        def body(i, carry):
            rows = pl.ds(i * BLK, BLK)
            y = jax.nn.relu(
                jnp.dot(gb_ref[rows, :], zb_ref[...],
                        preferred_element_type=jnp.float32)
                + b[None, :])
            if f32_ref is not None:
                f32_ref[rows, :] = y
            if store_ref is not None:
                nxt = y if skip_ref is None else (
                    y + skip_ref[rows, :].astype(jnp.float32))
                store_ref[rows, :] = nxt.astype(jnp.bfloat16)
            if final:
                o2_ref[rows, :] = y
                o3_ref[rows, :] = y + h_ref[rows, :]
            return carry

        jax.lax.fori_loop(0, N // BLK, body, 0, unroll=4)

    layer(t0_ref, wd_ref[1], bd_ref[1], store_ref=t1_ref)
    layer(t1_ref, wd_ref[2], bd_ref[2], store_ref=t2_ref)
    layer(t2_ref, wb_ref[...], bb_ref[...], store_ref=p0_ref,
          skip_ref=t2_ref)
    layer(p0_ref, wu_ref[0], bu_ref[0], store_ref=p1_ref,
          skip_ref=t1_ref, f32_ref=o0_ref)
    layer(p1_ref, wu_ref[1], bu_ref[1], store_ref=p0_ref,
          skip_ref=t0_ref, f32_ref=o1_ref)
    layer(p0_ref, wu_ref[2], bu_ref[2], final=True)


def kernel(g, h, W_down, b_down, W_up, b_up, W_bottom, b_bottom):
    out = pl.pallas_call(
        _unet_kernel,
        out_shape=tuple(
            jax.ShapeDtypeStruct((N, DIM), jnp.float32) for _ in range(4)),
        in_specs=[pl.BlockSpec(memory_space=pl.ANY)] + [
            pl.BlockSpec(memory_space=pltpu.VMEM) for _ in range(7)],
        scratch_shapes=(
            [pltpu.VMEM((N, N), jnp.bfloat16),
             pltpu.VMEM((2, CH, N), jnp.float32)]
            + [pltpu.VMEM((N, DIM), jnp.bfloat16) for _ in range(6)]
            + [pltpu.SemaphoreType.DMA((2,))]),
    )(g, h, W_down, b_down, W_up, b_up, W_bottom, b_bottom)
    return out


# speedup vs baseline: 1.0274x; 1.0274x over previous
import jax
import jax.numpy as jnp
from jax.experimental import pallas as pl
from jax.experimental.pallas import tpu as pltpu

N = 4096
DIM = 128
L = 3
CH = 256
BLK = 512


def _unet_kernel(g_hbm, h_ref, wd_ref, bd_ref, wu_ref, bu_ref, wb_ref, bb_ref,
                 o0_ref, o1_ref, o2_ref, o3_ref,
                 gb_ref, stage_ref, zb_ref, t0_ref, t1_ref, t2_ref,
                 p0_ref, p1_ref, sem):

    def g_dma(i, slot):
        return pltpu.make_async_copy(
            g_hbm.at[pl.ds(i * CH, CH), :], stage_ref.at[slot], sem.at[slot])

    zb_ref[...] = jnp.dot(
        h_ref[...].astype(jnp.bfloat16), wd_ref[0].astype(jnp.bfloat16),
        preferred_element_type=jnp.float32).astype(jnp.bfloat16)
    b0 = bd_ref[0]

    g_dma(0, 0).start()
    for i in range(N // CH):
        slot = i % 2
        if i + 1 < N // CH:
            g_dma(i + 1, 1 - slot).start()
        g_dma(i, slot).wait()
        gchunk = stage_ref[slot].astype(jnp.bfloat16)
        gb_ref[pl.ds(i * CH, CH), :] = gchunk
        y = jax.nn.relu(
            jnp.dot(gchunk, zb_ref[...], preferred_element_type=jnp.float32)
            + b0[None, :])
        t0_ref[pl.ds(i * CH, CH), :] = y.astype(jnp.bfloat16)

    def layer(x_ref, W, b, store_ref=None, skip_ref=None, f32_ref=None,
              final=False):
        zb_ref[...] = jnp.dot(
            x_ref[...], W.astype(jnp.bfloat16),
            preferred_element_type=jnp.float32).astype(jnp.bfloat16)

        def body(i, carry):
            rows = pl.ds(i * BLK, BLK)
            y = jax.nn.relu(
                jnp.dot(gb_ref[rows, :], zb_ref[...],
                        preferred_element_type=jnp.float32)
                + b[None, :])
            if f32_ref is not None:
                f32_ref[rows, :] = y
            if store_ref is not None:
                nxt = y if skip_ref is None else (
                    y + skip_ref[rows, :].astype(jnp.float32))
                store_ref[rows, :] = nxt.astype(jnp.bfloat16)
            if final:
                o2_ref[rows, :] = y
                o3_ref[rows, :] = y + h_ref[rows, :]
            return carry

        jax.lax.fori_loop(0, N // BLK, body, 0, unroll=4)

    layer(t0_ref, wd_ref[1], bd_ref[1], store_ref=t1_ref)
    layer(t1_ref, wd_ref[2], bd_ref[2], store_ref=t2_ref)
    layer(t2_ref, wb_ref[...], bb_ref[...], store_ref=p0_ref,
          skip_ref=t2_ref)
    layer(p0_ref, wu_ref[0], bu_ref[0], store_ref=p1_ref,
          skip_ref=t1_ref, f32_ref=o0_ref)
    layer(p1_ref, wu_ref[1], bu_ref[1], store_ref=p0_ref,
          skip_ref=t0_ref, f32_ref=o1_ref)
    layer(p0_ref, wu_ref[2], bu_ref[2], final=True)


def kernel(g, h, W_down, b_down, W_up, b_up, W_bottom, b_bottom):
    out = pl.pallas_call(
        _unet_kernel,
        out_shape=tuple(
            jax.ShapeDtypeStruct((N, DIM), jnp.float32) for _ in range(4)),
        in_specs=[pl.BlockSpec(memory_space=pl.ANY)] + [
            pl.BlockSpec(memory_space=pltpu.VMEM) for _ in range(7)],
        scratch_shapes=(
            [pltpu.VMEM((N, N), jnp.bfloat16),
             pltpu.VMEM((2, CH, N), jnp.float32)]
            + [pltpu.VMEM((N, DIM), jnp.bfloat16) for _ in range(6)]
            + [pltpu.SemaphoreType.DMA((2,))]),
    )(g, h, W_down, b_down, W_up, b_up, W_bottom, b_bottom)
    return out


# speedup vs baseline: 1.0711x; 1.0426x over previous
import jax
import jax.numpy as jnp
from jax.experimental import pallas as pl
from jax.experimental.pallas import tpu as pltpu

N = 4096
DIM = 128
L = 3
CH = 128
NSLOT = 4
BLK = 512


def _unet_kernel(g_hbm, h_ref, wd_ref, bd_ref, wu_ref, bu_ref, wb_ref, bb_ref,
                 o0_ref, o1_ref, o2_ref, o3_ref,
                 gb_ref, stage_ref, zb_ref, t0_ref, t1_ref, t2_ref,
                 p0_ref, p1_ref, sem):

    def g_dma(i, slot):
        return pltpu.make_async_copy(
            g_hbm.at[pl.ds(i * CH, CH), :], stage_ref.at[slot], sem.at[slot])

    zb_ref[...] = jnp.dot(
        h_ref[...].astype(jnp.bfloat16), wd_ref[0].astype(jnp.bfloat16),
        preferred_element_type=jnp.float32).astype(jnp.bfloat16)
    b0 = bd_ref[0]

    for j in range(NSLOT - 1):
        g_dma(j, j).start()
    for i in range(N // CH):
        slot = i % NSLOT
        if i + NSLOT - 1 < N // CH:
            g_dma(i + NSLOT - 1, (i + NSLOT - 1) % NSLOT).start()
        g_dma(i, slot).wait()
        gchunk = stage_ref[slot].astype(jnp.bfloat16)
        gb_ref[pl.ds(i * CH, CH), :] = gchunk
        y = jax.nn.relu(
            jnp.dot(gchunk, zb_ref[...], preferred_element_type=jnp.float32)
            + b0[None, :])
        t0_ref[pl.ds(i * CH, CH), :] = y.astype(jnp.bfloat16)

    def layer(x_ref, W, b, store_ref=None, skip_ref=None, f32_ref=None,
              final=False):
        zb_ref[...] = jnp.dot(
            x_ref[...], W.astype(jnp.bfloat16),
            preferred_element_type=jnp.float32).astype(jnp.bfloat16)

        def body(i, carry):
            rows = pl.ds(i * BLK, BLK)
            y = jax.nn.relu(
                jnp.dot(gb_ref[rows, :], zb_ref[...],
                        preferred_element_type=jnp.float32)
                + b[None, :])
            if f32_ref is not None:
                f32_ref[rows, :] = y
            if store_ref is not None:
                nxt = y if skip_ref is None else (
                    y + skip_ref[rows, :].astype(jnp.float32))
                store_ref[rows, :] = nxt.astype(jnp.bfloat16)
            if final:
                o2_ref[rows, :] = y
                o3_ref[rows, :] = y + h_ref[rows, :]
            return carry

        jax.lax.fori_loop(0, N // BLK, body, 0, unroll=4)

    layer(t0_ref, wd_ref[1], bd_ref[1], store_ref=t1_ref)
    layer(t1_ref, wd_ref[2], bd_ref[2], store_ref=t2_ref)
    layer(t2_ref, wb_ref[...], bb_ref[...], store_ref=p0_ref,
          skip_ref=t2_ref)
    layer(p0_ref, wu_ref[0], bu_ref[0], store_ref=p1_ref,
          skip_ref=t1_ref, f32_ref=o0_ref)
    layer(p1_ref, wu_ref[1], bu_ref[1], store_ref=p0_ref,
          skip_ref=t0_ref, f32_ref=o1_ref)
    layer(p0_ref, wu_ref[2], bu_ref[2], final=True)


def kernel(g, h, W_down, b_down, W_up, b_up, W_bottom, b_bottom):
    out = pl.pallas_call(
        _unet_kernel,
        out_shape=tuple(
            jax.ShapeDtypeStruct((N, DIM), jnp.float32) for _ in range(4)),
        in_specs=[pl.BlockSpec(memory_space=pl.ANY)] + [
            pl.BlockSpec(memory_space=pltpu.VMEM) for _ in range(7)],
        scratch_shapes=(
            [pltpu.VMEM((N, N), jnp.bfloat16),
             pltpu.VMEM((NSLOT, CH, N), jnp.float32)]
            + [pltpu.VMEM((N, DIM), jnp.bfloat16) for _ in range(6)]
            + [pltpu.SemaphoreType.DMA((NSLOT,))]),
    )(g, h, W_down, b_down, W_up, b_up, W_bottom, b_bottom)
    return out
